# baseline (speedup 1.0000x reference)
import jax
import jax.numpy as jnp
from jax import lax
from jax.experimental import pallas as pl
from jax.experimental.pallas import tpu as pltpu
from jax.experimental.pallas import tpu_sc as plsc

N = 10000
E = 320000
D = 128
G = 64

NC = 2
NS = 16
NW = NC * NS

EC = 80
ECH_PER_W = E // (EC * NW)
MPC = 80
MPCH = E // (NW * MPC)
RC = 80
N_RCHUNK = N // RC

RT = 624
RT_LAST = N - 15 * RT

_mesh = plsc.VectorSubcoreMesh(core_axis_name="c", subcore_axis_name="s")


def _stats_body(ei_hbm, ni_hbm, emb_hbm, on_hbm, zrow_hbm,
                x_out, degs0_out, degs1_out, degd0_out, degd1_out,
                gcnt0_out, gcnt1_out,
                ib0, ib1, ib2, ib3, nib0, nib1, ones_v, rows0_v, rows1_v,
                zbuf, is0, is1, is2, is3, nis0, nis1, ng0, ng1,
                degs_sh, degd_sh, gcnt_sh):
    c = lax.axis_index("c")
    s = lax.axis_index("s")
    w = s * NC + c

    ibs = [ib0, ib1, ib2, ib3]
    isem = [is0, is1, is2, is3]

    pltpu.sync_copy(on_hbm, ones_v)

    def zb(j, carry):
        zbuf[pl.ds(j * 16, 16)] = jnp.zeros((16,), jnp.float32)
        return carry

    lax.fori_loop(0, RT_LAST // 16, zb, 0)

    off = s * RT

    @pl.when(s < 15)
    def _():
        pltpu.sync_copy(zbuf.at[pl.ds(0, RT)], degs_sh.at[pl.ds(off, RT)])
        pltpu.sync_copy(zbuf.at[pl.ds(0, RT)], degd_sh.at[pl.ds(off, RT)])

    @pl.when(s == 15)
    def _():
        pltpu.sync_copy(zbuf, degs_sh.at[pl.ds(15 * RT, RT_LAST)])
        pltpu.sync_copy(zbuf, degd_sh.at[pl.ds(15 * RT, RT_LAST)])

    @pl.when(s < 4)
    def _():
        pltpu.sync_copy(zbuf.at[pl.ds(0, 16)], gcnt_sh.at[pl.ds(s * 16, 16)])

    plsc.subcore_barrier()

    def iload(k, b):
        pltpu.async_copy(ei_hbm.at[w, k], ibs[b], isem[b])

    def iwait(b):
        pltpu.make_async_copy(ei_hbm.at[w, 0], ibs[b], isem[b]).wait()

    def spair(b):
        pltpu.sync_copy(ones_v, degs_sh.at[ibs[b].at[0]], add=True)
        pltpu.sync_copy(ones_v, degd_sh.at[ibs[b].at[1]], add=True)

    iload(0, 0)
    iload(1, 1)
    iload(2, 2)
    iload(3, 3)

    def body(j, carry):
        k0 = 4 * j
        for b in range(4):
            k = k0 + b
            iwait(b)
            spair(b)

            @pl.when(k + 4 < MPCH)
            def _():
                pltpu.async_copy(ei_hbm.at[w, k + 4], ibs[b], isem[b])

        return carry

    lax.fori_loop(0, 31, body, 0)
    iwait(0)
    spair(0)

    def niload(ci, nib, sem):
        pltpu.async_copy(ni_hbm.at[ci], nib, sem)

    def niwait(nib, sem):
        pltpu.make_async_copy(ni_hbm.at[0], nib, sem).wait()

    def ngather(nib, rows, sem):
        pltpu.async_copy(emb_hbm.at[nib.at[0]], rows, sem)

    def ngwait(rows, sem):
        pltpu.make_async_copy(zrow_hbm, rows, sem).wait()

    def gscatter(nib):
        pltpu.sync_copy(ones_v, gcnt_sh.at[nib.at[1]], add=True)

    def xstore(ci, rows):
        pltpu.sync_copy(rows, x_out.at[pl.ds(ci * RC, RC)])

    c0 = w
    c1 = w + NW
    c2 = w + 2 * NW
    c3 = w + 3 * NW
    niload(c0, nib0, nis0)
    niload(c1, nib1, nis1)
    niwait(nib0, nis0)
    ngather(nib0, rows0_v, ng0)
    gscatter(nib0)
    niwait(nib1, nis1)
    ngather(nib1, rows1_v, ng1)
    gscatter(nib1)
    ngwait(rows0_v, ng0)
    xstore(c0, rows0_v)
    niload(c2, nib0, nis0)
    ngwait(rows1_v, ng1)
    xstore(c1, rows1_v)

    @pl.when(c3 < N_RCHUNK)
    def _():
        niload(c3, nib1, nis1)

    niwait(nib0, nis0)
    ngather(nib0, rows0_v, ng0)
    gscatter(nib0)

    @pl.when(c3 < N_RCHUNK)
    def _():
        niwait(nib1, nis1)
        ngather(nib1, rows1_v, ng1)
        gscatter(nib1)

    ngwait(rows0_v, ng0)
    xstore(c2, rows0_v)

    @pl.when(c3 < N_RCHUNK)
    def _():
        ngwait(rows1_v, ng1)
        xstore(c3, rows1_v)

    plsc.subcore_barrier()

    def emit(sh_ref, out0, out1):
        @pl.when(s < 15)
        def _():
            pltpu.sync_copy(sh_ref.at[pl.ds(off, RT)], zbuf.at[pl.ds(0, RT)])

            @pl.when(c == 0)
            def _():
                pltpu.sync_copy(zbuf.at[pl.ds(0, RT)], out0.at[pl.ds(off, RT)])

            @pl.when(c == 1)
            def _():
                pltpu.sync_copy(zbuf.at[pl.ds(0, RT)], out1.at[pl.ds(off, RT)])

        @pl.when(s == 15)
        def _():
            pltpu.sync_copy(sh_ref.at[pl.ds(15 * RT, RT_LAST)], zbuf)

            @pl.when(c == 0)
            def _():
                pltpu.sync_copy(zbuf, out0.at[pl.ds(15 * RT, RT_LAST)])

            @pl.when(c == 1)
            def _():
                pltpu.sync_copy(zbuf, out1.at[pl.ds(15 * RT, RT_LAST)])

    emit(degs_sh, degs0_out, degs1_out)
    emit(degd_sh, degd0_out, degd1_out)

    @pl.when(s < 4)
    def _():
        pltpu.sync_copy(gcnt_sh.at[pl.ds(s * 16, 16)], zbuf.at[pl.ds(0, 16)])

        @pl.when(c == 0)
        def _():
            pltpu.sync_copy(zbuf.at[pl.ds(0, 16)],
                            gcnt0_out.at[pl.ds(s * 16, 16)])

        @pl.when(c == 1)
        def _():
            pltpu.sync_copy(zbuf.at[pl.ds(0, 16)],
                            gcnt1_out.at[pl.ds(s * 16, 16)])


_stats_call = pl.kernel(
    _stats_body,
    out_type=(
        jax.ShapeDtypeStruct((N, D), jnp.float32),
        jax.ShapeDtypeStruct((N,), jnp.float32),
        jax.ShapeDtypeStruct((N,), jnp.float32),
        jax.ShapeDtypeStruct((N,), jnp.float32),
        jax.ShapeDtypeStruct((N,), jnp.float32),
        jax.ShapeDtypeStruct((G,), jnp.float32),
        jax.ShapeDtypeStruct((G,), jnp.float32),
    ),
    mesh=_mesh,
    scratch_types=(
        [pltpu.VMEM((2, MPC), jnp.int32)] * 4
        + [pltpu.VMEM((2, RC), jnp.int32)] * 2
        + [pltpu.VMEM((MPC,), jnp.float32)]
        + [pltpu.VMEM((RC, D), jnp.float32)] * 2
        + [pltpu.VMEM((RT_LAST,), jnp.float32)]
        + [pltpu.SemaphoreType.DMA] * 8
        + [pltpu.VMEM_SHARED((N,), jnp.float32)] * 2
        + [pltpu.VMEM_SHARED((G,), jnp.float32)]
    ),
)


def _mp_body(xs_hbm, ei_hbm, znd_hbm, zrow_hbm, agg_out,
             ib0, ib1, ib2, ib3, rows0, rows1, rows2, rows3,
             is0, is1, is2, is3, g0, g1, g2, g3, s0, s1, s2, s3, agg_sh):
    c = lax.axis_index("c")
    s = lax.axis_index("s")
    w = s * NC + c

    ibs = [ib0, ib1, ib2, ib3]
    rows = [rows0, rows1, rows2, rows3]
    isem = [is0, is1, is2, is3]
    gsem = [g0, g1, g2, g3]
    ssem = [s0, s1, s2, s3]

    roff = s * RT

    @pl.when(s < 15)
    def _():
        pltpu.sync_copy(znd_hbm.at[pl.ds(roff, RT)],
                        agg_sh.at[pl.ds(roff, RT)])

    @pl.when(s == 15)
    def _():
        pltpu.sync_copy(znd_hbm.at[pl.ds(15 * RT, RT_LAST)],
                        agg_sh.at[pl.ds(15 * RT, RT_LAST)])

    plsc.subcore_barrier()

    def iload(k, b):
        pltpu.async_copy(ei_hbm.at[w, k], ibs[b], isem[b])

    def iwait(b):
        pltpu.make_async_copy(ei_hbm.at[w, 0], ibs[b], isem[b]).wait()

    def gather(b):
        pltpu.async_copy(xs_hbm.at[ibs[b].at[0]], rows[b], gsem[b])

    def gwait(b):
        pltpu.make_async_copy(zrow_hbm, rows[b], gsem[b]).wait()

    def sstart(b):
        pltpu.async_copy(rows[b], agg_sh.at[ibs[b].at[1]], ssem[b], add=True)

    def swait(b):
        pltpu.make_async_copy(zrow_hbm, rows[b], ssem[b]).wait()

    iload(0, 0)
    iload(1, 1)
    iwait(0)
    gather(0)

    iload(2, 2)
    iwait(1)
    gather(1)
    gwait(0)
    sstart(0)
    iload(3, 3)
    iwait(2)
    gather(2)
    gwait(1)
    sstart(1)

    def body(j, carry):
        k0 = 2 + 4 * j
        for b4 in range(4):
            k = k0 + b4
            bk = (2 + b4) % 4
            swait((bk + 2) % 4)
            pltpu.async_copy(ei_hbm.at[w, k + 2], ibs[(bk + 2) % 4],
                             isem[(bk + 2) % 4])
            iwait((bk + 1) % 4)
            gather((bk + 1) % 4)
            gwait(bk)
            sstart(bk)
        return carry

    lax.fori_loop(0, 30, body, 0)

    swait(0)
    iload(124, 0)
    iwait(3)
    gather(3)
    gwait(2)
    sstart(2)
    swait(1)
    iwait(0)
    gather(0)
    gwait(3)
    sstart(3)
    swait(2)
    gwait(0)
    sstart(0)
    swait(3)
    swait(0)

    plsc.subcore_barrier()

    @pl.when(s < 15)
    def _():
        pltpu.sync_copy(agg_sh.at[pl.ds(roff, RT)],
                        agg_out.at[c, pl.ds(roff, RT)])

    @pl.when(s == 15)
    def _():
        pltpu.sync_copy(agg_sh.at[pl.ds(15 * RT, RT_LAST)],
                        agg_out.at[c, pl.ds(15 * RT, RT_LAST)])


_mp_call = pl.kernel(
    _mp_body,
    out_type=jax.ShapeDtypeStruct((NC, N, D), jnp.float32),
    mesh=_mesh,
    scratch_types=(
        [pltpu.VMEM((2, MPC), jnp.int32)] * 4
        + [pltpu.VMEM((MPC, D), jnp.float32)] * 4
        + [pltpu.SemaphoreType.DMA] * 12
        + [pltpu.VMEM_SHARED((N, D), jnp.float32)]
    ),
)


BR = 1000


def _prep_tc(x_ref, dgs0_ref, dgs1_ref, dgd0_ref, dgd1_ref,
             xs_ref, ns_ref, nd_ref):
    ns = lax.rsqrt(jnp.clip(dgs0_ref[...] + dgs1_ref[...], 1.0, None))
    nd = lax.rsqrt(jnp.clip(dgd0_ref[...] + dgd1_ref[...], 1.0, None))
    ns_ref[...] = ns
    nd_ref[...] = nd
    xs_ref[...] = x_ref[...] * ns


def _prep_call(x, degs0, degs1, degd0, degd1):
    col = lambda a: a.reshape(N, 1)
    return pl.pallas_call(
        _prep_tc,
        grid=(N // BR,),
        in_specs=[
            pl.BlockSpec((BR, D), lambda i: (i, 0)),
            pl.BlockSpec((BR, 1), lambda i: (i, 0)),
            pl.BlockSpec((BR, 1), lambda i: (i, 0)),
            pl.BlockSpec((BR, 1), lambda i: (i, 0)),
            pl.BlockSpec((BR, 1), lambda i: (i, 0)),
        ],
        out_specs=[
            pl.BlockSpec((BR, D), lambda i: (i, 0)),
            pl.BlockSpec((BR, 1), lambda i: (i, 0)),
            pl.BlockSpec((BR, 1), lambda i: (i, 0)),
        ],
        out_shape=[
            jax.ShapeDtypeStruct((N, D), jnp.float32),
            jax.ShapeDtypeStruct((N, 1), jnp.float32),
            jax.ShapeDtypeStruct((N, 1), jnp.float32),
        ],
    )(x, col(degs0), col(degs1), col(degd0), col(degd1))


def _layer_tc(agg_ref, ns_ref, nd_ref, xin_ref, w_ref, b_ref, g_ref, be_ref,
              xo_ref, xso_ref):
    a = (agg_ref[0] + agg_ref[1]) * nd_ref[...]
    y = jnp.dot(a, w_ref[...], preferred_element_type=jnp.float32)
    t = jnp.maximum(g_ref[...] * (y + b_ref[...]) + be_ref[...], 0.0)
    xo = xin_ref[...] + t
    xo_ref[...] = xo
    xso_ref[...] = xo * ns_ref[...]


def _layer_call(agg, ns, nd, x_in, W, b, gamma, beta):
    return pl.pallas_call(
        _layer_tc,
        grid=(N // BR,),
        in_specs=[
            pl.BlockSpec((NC, BR, D), lambda i: (0, i, 0)),
            pl.BlockSpec((BR, 1), lambda i: (i, 0)),
            pl.BlockSpec((BR, 1), lambda i: (i, 0)),
            pl.BlockSpec((BR, D), lambda i: (i, 0)),
            pl.BlockSpec((D, D), lambda i: (0, 0)),
            pl.BlockSpec((1, D), lambda i: (0, 0)),
            pl.BlockSpec((1, D), lambda i: (0, 0)),
            pl.BlockSpec((1, D), lambda i: (0, 0)),
        ],
        out_specs=[
            pl.BlockSpec((BR, D), lambda i: (i, 0)),
            pl.BlockSpec((BR, D), lambda i: (i, 0)),
        ],
        out_shape=[
            jax.ShapeDtypeStruct((N, D), jnp.float32),
            jax.ShapeDtypeStruct((N, D), jnp.float32),
        ],
    )(agg, ns, nd, x_in, W, b.reshape(1, D), gamma.reshape(1, D),
      beta.reshape(1, D))


def _final_tc(agg_ref, nd_ref, xin_ref, w_ref, b_ref, g_ref, be_ref,
              gidc_ref, gc0_ref, gc1_ref, w1_ref, b1_ref, w2_ref, b2_ref,
              w3_ref, b3_ref, y_ref, acc_ref):
    i = pl.program_id(0)
    a = (agg_ref[0] + agg_ref[1]) * nd_ref[...]
    y = jnp.dot(a, w_ref[...], preferred_element_type=jnp.float32)
    t = jnp.maximum(g_ref[...] * (y + b_ref[...]) + be_ref[...], 0.0)
    xo = xin_ref[...] + t
    sel = (gidc_ref[...] ==
           lax.broadcasted_iota(jnp.int32, (1, G), 1)).astype(jnp.float32)
    contrib = lax.dot_general(sel, xo, (((0,), (0,)), ((), ())),
                              preferred_element_type=jnp.float32,
                              precision=lax.Precision.HIGHEST)

    @pl.when(i == 0)
    def _():
        acc_ref[...] = contrib

    @pl.when(i > 0)
    def _():
        acc_ref[...] = acc_ref[...] + contrib

    @pl.when(i == N // BR - 1)
    def _():
        cnt = jnp.clip(gc0_ref[...] + gc1_ref[...], 1.0, None)
        hg = acc_ref[...] / cnt
        z = jnp.maximum(jnp.dot(hg, w1_ref[...],
                                preferred_element_type=jnp.float32)
                        + b1_ref[...], 0.0)
        z = jnp.maximum(jnp.dot(z, w2_ref[...],
                                preferred_element_type=jnp.float32)
                        + b2_ref[...], 0.0)
        y_ref[...] = jnp.dot(z, w3_ref[...],
                             preferred_element_type=jnp.float32) + b3_ref[...]


def _final_call(agg, nd, x_in, W, b, gamma, beta, gidc, gcnt0, gcnt1,
                W1, b1, W2, b2, W3, b3):
    return pl.pallas_call(
        _final_tc,
        grid=(N // BR,),
        in_specs=[
            pl.BlockSpec((NC, BR, D), lambda i: (0, i, 0)),
            pl.BlockSpec((BR, 1), lambda i: (i, 0)),
            pl.BlockSpec((BR, D), lambda i: (i, 0)),
            pl.BlockSpec((D, D), lambda i: (0, 0)),
            pl.BlockSpec((1, D), lambda i: (0, 0)),
            pl.BlockSpec((1, D), lambda i: (0, 0)),
            pl.BlockSpec((1, D), lambda i: (0, 0)),
            pl.BlockSpec((BR, 1), lambda i: (i, 0)),
            pl.BlockSpec((G, 1), lambda i: (0, 0)),
            pl.BlockSpec((G, 1), lambda i: (0, 0)),
            pl.BlockSpec((D, D // 2), lambda i: (0, 0)),
            pl.BlockSpec((1, D // 2), lambda i: (0, 0)),
            pl.BlockSpec((D // 2, D // 4), lambda i: (0, 0)),
            pl.BlockSpec((1, D // 4), lambda i: (0, 0)),
            pl.BlockSpec((D // 4, 1), lambda i: (0, 0)),
            pl.BlockSpec((1, 1), lambda i: (0, 0)),
        ],
        out_specs=pl.BlockSpec((G, 1), lambda i: (0, 0)),
        out_shape=jax.ShapeDtypeStruct((G, 1), jnp.float32),
        scratch_shapes=[pltpu.VMEM((G, D), jnp.float32)],
    )(agg, nd, x_in, W, b.reshape(1, D), gamma.reshape(1, D),
      beta.reshape(1, D), gidc, gcnt0.reshape(G, 1), gcnt1.reshape(G, 1),
      W1, b1.reshape(1, D // 2), W2, b2.reshape(1, D // 4), W3,
      b3.reshape(1, 1))


def kernel(h, edge_index, e, node_graph_ids, emb, Ws, bs, gammas, betas,
           W1, b1, W2, b2, W3, b3):
    src = edge_index[0].astype(jnp.int32)
    dst = edge_index[1].astype(jnp.int32)
    h32 = h.astype(jnp.int32)
    gid = node_graph_ids.astype(jnp.int32)
    znd = jnp.zeros((N, D), jnp.float32)
    on = jnp.ones((EC,), jnp.float32)

    ei = jnp.stack([src.reshape(NW, MPCH, MPC),
                    dst.reshape(NW, MPCH, MPC)], axis=2)
    ni = jnp.stack([h32.reshape(N_RCHUNK, RC),
                    gid.reshape(N_RCHUNK, RC)], axis=1)
    zrow = jnp.zeros((MPC, D), jnp.float32)
    x, degs0, degs1, degd0, degd1, gcnt0, gcnt1 = _stats_call(
        ei, ni, emb, on, zrow)
    xs, ns, nd = _prep_call(x, degs0, degs1, degd0, degd1)
    for l in range(Ws.shape[0] - 1):
        agg = _mp_call(xs, ei, znd, zrow)
        x, xs = _layer_call(agg, ns, nd, x, Ws[l], bs[l], gammas[l], betas[l])
    agg = _mp_call(xs, ei, znd, zrow)
    L = Ws.shape[0] - 1
    return _final_call(agg, nd, x, Ws[L], bs[L], gammas[L], betas[L],
                       gid.reshape(N, 1), gcnt0, gcnt1, W1, b1, W2, b2, W3, b3)


# speedup vs baseline: 1.0073x; 1.0073x over previous
import jax
import jax.numpy as jnp
from jax import lax
from jax.experimental import pallas as pl
from jax.experimental.pallas import tpu as pltpu
from jax.experimental.pallas import tpu_sc as plsc

N = 10000
E = 320000
D = 128
G = 64

NC = 2
NS = 16
NW = NC * NS

EC = 80
ECH_PER_W = E // (EC * NW)
MPC = 80
MPCH = E // (NW * MPC)
RC = 80
N_RCHUNK = N // RC

RT = 624
RT_LAST = N - 15 * RT

_mesh = plsc.VectorSubcoreMesh(core_axis_name="c", subcore_axis_name="s")


def _stats_body(ei_hbm, ni_hbm, emb_hbm, on_hbm, zrow_hbm,
                x_out, degs0_out, degs1_out, degd0_out, degd1_out,
                gcnt0_out, gcnt1_out,
                ib0, ib1, ib2, ib3, nib0, nib1, ones_v, rows0_v, rows1_v,
                zbuf, is0, is1, is2, is3, nis0, nis1, ng0, ng1,
                degs_sh, degd_sh, gcnt_sh):
    c = lax.axis_index("c")
    s = lax.axis_index("s")
    w = s * NC + c

    ibs = [ib0, ib1, ib2, ib3]
    isem = [is0, is1, is2, is3]

    pltpu.sync_copy(on_hbm, ones_v)

    def zb(j, carry):
        zbuf[pl.ds(j * 16, 16)] = jnp.zeros((16,), jnp.float32)
        return carry

    lax.fori_loop(0, RT_LAST // 16, zb, 0)

    off = s * RT

    @pl.when(s < 15)
    def _():
        pltpu.sync_copy(zbuf.at[pl.ds(0, RT)], degs_sh.at[pl.ds(off, RT)])
        pltpu.sync_copy(zbuf.at[pl.ds(0, RT)], degd_sh.at[pl.ds(off, RT)])

    @pl.when(s == 15)
    def _():
        pltpu.sync_copy(zbuf, degs_sh.at[pl.ds(15 * RT, RT_LAST)])
        pltpu.sync_copy(zbuf, degd_sh.at[pl.ds(15 * RT, RT_LAST)])

    @pl.when(s < 4)
    def _():
        pltpu.sync_copy(zbuf.at[pl.ds(0, 16)], gcnt_sh.at[pl.ds(s * 16, 16)])

    plsc.subcore_barrier()

    def iload(k, b):
        pltpu.async_copy(ei_hbm.at[w, k], ibs[b], isem[b])

    def iwait(b):
        pltpu.make_async_copy(ei_hbm.at[w, 0], ibs[b], isem[b]).wait()

    def spair(b):
        pltpu.sync_copy(ones_v, degs_sh.at[ibs[b].at[0]], add=True)
        pltpu.sync_copy(ones_v, degd_sh.at[ibs[b].at[1]], add=True)

    iload(0, 0)
    iload(1, 1)
    iload(2, 2)
    iload(3, 3)

    def body(j, carry):
        k0 = 4 * j
        for b in range(4):
            k = k0 + b
            iwait(b)
            spair(b)

            @pl.when(k + 4 < MPCH)
            def _():
                pltpu.async_copy(ei_hbm.at[w, k + 4], ibs[b], isem[b])

        return carry

    lax.fori_loop(0, 31, body, 0)
    iwait(0)
    spair(0)

    def niload(ci, nib, sem):
        pltpu.async_copy(ni_hbm.at[ci], nib, sem)

    def niwait(nib, sem):
        pltpu.make_async_copy(ni_hbm.at[0], nib, sem).wait()

    def ngather(nib, rows, sem):
        pltpu.async_copy(emb_hbm.at[nib.at[0]], rows, sem)

    def ngwait(rows, sem):
        pltpu.make_async_copy(zrow_hbm, rows, sem).wait()

    def gscatter(nib):
        pltpu.sync_copy(ones_v, gcnt_sh.at[nib.at[1]], add=True)

    def xstore(ci, rows):
        pltpu.sync_copy(rows, x_out.at[pl.ds(ci * RC, RC)])

    c0 = w
    c1 = w + NW
    c2 = w + 2 * NW
    c3 = w + 3 * NW
    niload(c0, nib0, nis0)
    niload(c1, nib1, nis1)
    niwait(nib0, nis0)
    ngather(nib0, rows0_v, ng0)
    gscatter(nib0)
    niwait(nib1, nis1)
    ngather(nib1, rows1_v, ng1)
    gscatter(nib1)
    ngwait(rows0_v, ng0)
    xstore(c0, rows0_v)
    niload(c2, nib0, nis0)
    ngwait(rows1_v, ng1)
    xstore(c1, rows1_v)

    @pl.when(c3 < N_RCHUNK)
    def _():
        niload(c3, nib1, nis1)

    niwait(nib0, nis0)
    ngather(nib0, rows0_v, ng0)
    gscatter(nib0)

    @pl.when(c3 < N_RCHUNK)
    def _():
        niwait(nib1, nis1)
        ngather(nib1, rows1_v, ng1)
        gscatter(nib1)

    ngwait(rows0_v, ng0)
    xstore(c2, rows0_v)

    @pl.when(c3 < N_RCHUNK)
    def _():
        ngwait(rows1_v, ng1)
        xstore(c3, rows1_v)

    plsc.subcore_barrier()

    def emit(sh_ref, out0, out1):
        @pl.when(s < 15)
        def _():
            pltpu.sync_copy(sh_ref.at[pl.ds(off, RT)], zbuf.at[pl.ds(0, RT)])

            @pl.when(c == 0)
            def _():
                pltpu.sync_copy(zbuf.at[pl.ds(0, RT)], out0.at[pl.ds(off, RT)])

            @pl.when(c == 1)
            def _():
                pltpu.sync_copy(zbuf.at[pl.ds(0, RT)], out1.at[pl.ds(off, RT)])

        @pl.when(s == 15)
        def _():
            pltpu.sync_copy(sh_ref.at[pl.ds(15 * RT, RT_LAST)], zbuf)

            @pl.when(c == 0)
            def _():
                pltpu.sync_copy(zbuf, out0.at[pl.ds(15 * RT, RT_LAST)])

            @pl.when(c == 1)
            def _():
                pltpu.sync_copy(zbuf, out1.at[pl.ds(15 * RT, RT_LAST)])

    emit(degs_sh, degs0_out, degs1_out)
    emit(degd_sh, degd0_out, degd1_out)

    @pl.when(s < 4)
    def _():
        pltpu.sync_copy(gcnt_sh.at[pl.ds(s * 16, 16)], zbuf.at[pl.ds(0, 16)])

        @pl.when(c == 0)
        def _():
            pltpu.sync_copy(zbuf.at[pl.ds(0, 16)],
                            gcnt0_out.at[pl.ds(s * 16, 16)])

        @pl.when(c == 1)
        def _():
            pltpu.sync_copy(zbuf.at[pl.ds(0, 16)],
                            gcnt1_out.at[pl.ds(s * 16, 16)])


_stats_call = pl.kernel(
    _stats_body,
    out_type=(
        jax.ShapeDtypeStruct((N, D), jnp.float32),
        jax.ShapeDtypeStruct((N,), jnp.float32),
        jax.ShapeDtypeStruct((N,), jnp.float32),
        jax.ShapeDtypeStruct((N,), jnp.float32),
        jax.ShapeDtypeStruct((N,), jnp.float32),
        jax.ShapeDtypeStruct((G,), jnp.float32),
        jax.ShapeDtypeStruct((G,), jnp.float32),
    ),
    mesh=_mesh,
    scratch_types=(
        [pltpu.VMEM((2, MPC), jnp.int32)] * 4
        + [pltpu.VMEM((2, RC), jnp.int32)] * 2
        + [pltpu.VMEM((MPC,), jnp.float32)]
        + [pltpu.VMEM((RC, D), jnp.float32)] * 2
        + [pltpu.VMEM((RT_LAST,), jnp.float32)]
        + [pltpu.SemaphoreType.DMA] * 8
        + [pltpu.VMEM_SHARED((N,), jnp.float32)] * 2
        + [pltpu.VMEM_SHARED((G,), jnp.float32)]
    ),
)


def _mp_body(xs_hbm, ei_hbm, znd_hbm, zrow_hbm, agg_out,
             ib0, ib1, ib2, ib3, rows0, rows1, rows2, rows3,
             is0, is1, is2, is3, g0, g1, g2, g3, s0, s1, s2, s3, agg_sh):
    c = lax.axis_index("c")
    s = lax.axis_index("s")
    w = s * NC + c

    ibs = [ib0, ib1, ib2, ib3]
    rows = [rows0, rows1, rows2, rows3]
    isem = [is0, is1, is2, is3]
    gsem = [g0, g1, g2, g3]
    ssem = [s0, s1, s2, s3]

    roff = s * RT

    def iload(k, b):
        pltpu.async_copy(ei_hbm.at[w, k], ibs[b], isem[b])

    def iwait(b):
        pltpu.make_async_copy(ei_hbm.at[w, 0], ibs[b], isem[b]).wait()

    def gather(b):
        pltpu.async_copy(xs_hbm.at[ibs[b].at[0]], rows[b], gsem[b])

    def gwait(b):
        pltpu.make_async_copy(zrow_hbm, rows[b], gsem[b]).wait()

    def sstart(b):
        pltpu.async_copy(rows[b], agg_sh.at[ibs[b].at[1]], ssem[b], add=True)

    def swait(b):
        pltpu.make_async_copy(zrow_hbm, rows[b], ssem[b]).wait()

    iload(0, 0)
    iload(1, 1)
    iwait(0)
    gather(0)
    iload(2, 2)
    iwait(1)
    gather(1)

    @pl.when(s < 15)
    def _():
        pltpu.sync_copy(znd_hbm.at[pl.ds(roff, RT)],
                        agg_sh.at[pl.ds(roff, RT)])

    @pl.when(s == 15)
    def _():
        pltpu.sync_copy(znd_hbm.at[pl.ds(15 * RT, RT_LAST)],
                        agg_sh.at[pl.ds(15 * RT, RT_LAST)])

    plsc.subcore_barrier()

    gwait(0)
    sstart(0)
    iload(3, 3)
    iwait(2)
    gather(2)
    gwait(1)
    sstart(1)

    def body(j, carry):
        k0 = 2 + 4 * j
        for b4 in range(4):
            k = k0 + b4
            bk = (2 + b4) % 4
            swait((bk + 2) % 4)
            pltpu.async_copy(ei_hbm.at[w, k + 2], ibs[(bk + 2) % 4],
                             isem[(bk + 2) % 4])
            iwait((bk + 1) % 4)
            gather((bk + 1) % 4)
            gwait(bk)
            sstart(bk)
        return carry

    lax.fori_loop(0, 30, body, 0)

    swait(0)
    iload(124, 0)
    iwait(3)
    gather(3)
    gwait(2)
    sstart(2)
    swait(1)
    iwait(0)
    gather(0)
    gwait(3)
    sstart(3)
    swait(2)
    gwait(0)
    sstart(0)
    swait(3)
    swait(0)

    plsc.subcore_barrier()

    @pl.when(s < 15)
    def _():
        pltpu.sync_copy(agg_sh.at[pl.ds(roff, RT)],
                        agg_out.at[c, pl.ds(roff, RT)])

    @pl.when(s == 15)
    def _():
        pltpu.sync_copy(agg_sh.at[pl.ds(15 * RT, RT_LAST)],
                        agg_out.at[c, pl.ds(15 * RT, RT_LAST)])


_mp_call = pl.kernel(
    _mp_body,
    out_type=jax.ShapeDtypeStruct((NC, N, D), jnp.float32),
    mesh=_mesh,
    scratch_types=(
        [pltpu.VMEM((2, MPC), jnp.int32)] * 4
        + [pltpu.VMEM((MPC, D), jnp.float32)] * 4
        + [pltpu.SemaphoreType.DMA] * 12
        + [pltpu.VMEM_SHARED((N, D), jnp.float32)]
    ),
)


BR = 1000


def _prep_tc(x_ref, dgs0_ref, dgs1_ref, dgd0_ref, dgd1_ref,
             xs_ref, ns_ref, nd_ref):
    ns = lax.rsqrt(jnp.clip(dgs0_ref[...] + dgs1_ref[...], 1.0, None))
    nd = lax.rsqrt(jnp.clip(dgd0_ref[...] + dgd1_ref[...], 1.0, None))
    ns_ref[...] = ns
    nd_ref[...] = nd
    xs_ref[...] = x_ref[...] * ns


def _prep_call(x, degs0, degs1, degd0, degd1):
    col = lambda a: a.reshape(N, 1)
    return pl.pallas_call(
        _prep_tc,
        grid=(N // BR,),
        in_specs=[
            pl.BlockSpec((BR, D), lambda i: (i, 0)),
            pl.BlockSpec((BR, 1), lambda i: (i, 0)),
            pl.BlockSpec((BR, 1), lambda i: (i, 0)),
            pl.BlockSpec((BR, 1), lambda i: (i, 0)),
            pl.BlockSpec((BR, 1), lambda i: (i, 0)),
        ],
        out_specs=[
            pl.BlockSpec((BR, D), lambda i: (i, 0)),
            pl.BlockSpec((BR, 1), lambda i: (i, 0)),
            pl.BlockSpec((BR, 1), lambda i: (i, 0)),
        ],
        out_shape=[
            jax.ShapeDtypeStruct((N, D), jnp.float32),
            jax.ShapeDtypeStruct((N, 1), jnp.float32),
            jax.ShapeDtypeStruct((N, 1), jnp.float32),
        ],
    )(x, col(degs0), col(degs1), col(degd0), col(degd1))


def _layer_tc(agg_ref, ns_ref, nd_ref, xin_ref, w_ref, b_ref, g_ref, be_ref,
              xo_ref, xso_ref):
    a = (agg_ref[0] + agg_ref[1]) * nd_ref[...]
    y = jnp.dot(a, w_ref[...], preferred_element_type=jnp.float32)
    t = jnp.maximum(g_ref[...] * (y + b_ref[...]) + be_ref[...], 0.0)
    xo = xin_ref[...] + t
    xo_ref[...] = xo
    xso_ref[...] = xo * ns_ref[...]


def _layer_call(agg, ns, nd, x_in, W, b, gamma, beta):
    return pl.pallas_call(
        _layer_tc,
        grid=(N // BR,),
        in_specs=[
            pl.BlockSpec((NC, BR, D), lambda i: (0, i, 0)),
            pl.BlockSpec((BR, 1), lambda i: (i, 0)),
            pl.BlockSpec((BR, 1), lambda i: (i, 0)),
            pl.BlockSpec((BR, D), lambda i: (i, 0)),
            pl.BlockSpec((D, D), lambda i: (0, 0)),
            pl.BlockSpec((1, D), lambda i: (0, 0)),
            pl.BlockSpec((1, D), lambda i: (0, 0)),
            pl.BlockSpec((1, D), lambda i: (0, 0)),
        ],
        out_specs=[
            pl.BlockSpec((BR, D), lambda i: (i, 0)),
            pl.BlockSpec((BR, D), lambda i: (i, 0)),
        ],
        out_shape=[
            jax.ShapeDtypeStruct((N, D), jnp.float32),
            jax.ShapeDtypeStruct((N, D), jnp.float32),
        ],
    )(agg, ns, nd, x_in, W, b.reshape(1, D), gamma.reshape(1, D),
      beta.reshape(1, D))


def _final_tc(agg_ref, nd_ref, xin_ref, w_ref, b_ref, g_ref, be_ref,
              gidc_ref, gc0_ref, gc1_ref, w1_ref, b1_ref, w2_ref, b2_ref,
              w3_ref, b3_ref, y_ref, acc_ref):
    i = pl.program_id(0)
    a = (agg_ref[0] + agg_ref[1]) * nd_ref[...]
    y = jnp.dot(a, w_ref[...], preferred_element_type=jnp.float32)
    t = jnp.maximum(g_ref[...] * (y + b_ref[...]) + be_ref[...], 0.0)
    xo = xin_ref[...] + t
    sel = (gidc_ref[...] ==
           lax.broadcasted_iota(jnp.int32, (1, G), 1)).astype(jnp.float32)
    contrib = lax.dot_general(sel, xo, (((0,), (0,)), ((), ())),
                              preferred_element_type=jnp.float32,
                              precision=lax.Precision.HIGHEST)

    @pl.when(i == 0)
    def _():
        acc_ref[...] = contrib

    @pl.when(i > 0)
    def _():
        acc_ref[...] = acc_ref[...] + contrib

    @pl.when(i == N // BR - 1)
    def _():
        cnt = jnp.clip(gc0_ref[...] + gc1_ref[...], 1.0, None)
        hg = acc_ref[...] / cnt
        z = jnp.maximum(jnp.dot(hg, w1_ref[...],
                                preferred_element_type=jnp.float32)
                        + b1_ref[...], 0.0)
        z = jnp.maximum(jnp.dot(z, w2_ref[...],
                                preferred_element_type=jnp.float32)
                        + b2_ref[...], 0.0)
        y_ref[...] = jnp.dot(z, w3_ref[...],
                             preferred_element_type=jnp.float32) + b3_ref[...]


def _final_call(agg, nd, x_in, W, b, gamma, beta, gidc, gcnt0, gcnt1,
                W1, b1, W2, b2, W3, b3):
    return pl.pallas_call(
        _final_tc,
        grid=(N // BR,),
        in_specs=[
            pl.BlockSpec((NC, BR, D), lambda i: (0, i, 0)),
            pl.BlockSpec((BR, 1), lambda i: (i, 0)),
            pl.BlockSpec((BR, D), lambda i: (i, 0)),
            pl.BlockSpec((D, D), lambda i: (0, 0)),
            pl.BlockSpec((1, D), lambda i: (0, 0)),
            pl.BlockSpec((1, D), lambda i: (0, 0)),
            pl.BlockSpec((1, D), lambda i: (0, 0)),
            pl.BlockSpec((BR, 1), lambda i: (i, 0)),
            pl.BlockSpec((G, 1), lambda i: (0, 0)),
            pl.BlockSpec((G, 1), lambda i: (0, 0)),
            pl.BlockSpec((D, D // 2), lambda i: (0, 0)),
            pl.BlockSpec((1, D // 2), lambda i: (0, 0)),
            pl.BlockSpec((D // 2, D // 4), lambda i: (0, 0)),
            pl.BlockSpec((1, D // 4), lambda i: (0, 0)),
            pl.BlockSpec((D // 4, 1), lambda i: (0, 0)),
            pl.BlockSpec((1, 1), lambda i: (0, 0)),
        ],
        out_specs=pl.BlockSpec((G, 1), lambda i: (0, 0)),
        out_shape=jax.ShapeDtypeStruct((G, 1), jnp.float32),
        scratch_shapes=[pltpu.VMEM((G, D), jnp.float32)],
    )(agg, nd, x_in, W, b.reshape(1, D), gamma.reshape(1, D),
      beta.reshape(1, D), gidc, gcnt0.reshape(G, 1), gcnt1.reshape(G, 1),
      W1, b1.reshape(1, D // 2), W2, b2.reshape(1, D // 4), W3,
      b3.reshape(1, 1))


def kernel(h, edge_index, e, node_graph_ids, emb, Ws, bs, gammas, betas,
           W1, b1, W2, b2, W3, b3):
    src = edge_index[0].astype(jnp.int32)
    dst = edge_index[1].astype(jnp.int32)
    h32 = h.astype(jnp.int32)
    gid = node_graph_ids.astype(jnp.int32)
    znd = jnp.zeros((N, D), jnp.float32)
    on = jnp.ones((EC,), jnp.float32)

    ei = jnp.stack([src.reshape(NW, MPCH, MPC),
                    dst.reshape(NW, MPCH, MPC)], axis=2)
    ni = jnp.stack([h32.reshape(N_RCHUNK, RC),
                    gid.reshape(N_RCHUNK, RC)], axis=1)
    zrow = jnp.zeros((MPC, D), jnp.float32)
    x, degs0, degs1, degd0, degd1, gcnt0, gcnt1 = _stats_call(
        ei, ni, emb, on, zrow)
    xs, ns, nd = _prep_call(x, degs0, degs1, degd0, degd1)
    for l in range(Ws.shape[0] - 1):
        agg = _mp_call(xs, ei, znd, zrow)
        x, xs = _layer_call(agg, ns, nd, x, Ws[l], bs[l], gammas[l], betas[l])
    agg = _mp_call(xs, ei, znd, zrow)
    L = Ws.shape[0] - 1
    return _final_call(agg, nd, x, Ws[L], bs[L], gammas[L], betas[L],
                       gid.reshape(N, 1), gcnt0, gcnt1, W1, b1, W2, b2, W3, b3)
